# baseline (speedup 1.0000x reference)
import jax
import jax.numpy as jnp
from jax import lax
from jax.experimental import pallas as pl
from jax.experimental.pallas import tpu as pltpu
from jax.experimental.pallas import tpu_sc as plsc

N = 10000
E = 320000
D = 128

NC = 2
NS = 16
NW = NC * NS
NP = 10240
RPS = NP // NS
K = 128
EWP = NP
CH = EWP // K
SECR = 8
SECS = CH // SECR
HW = 16


def _mesh():
    return plsc.VectorSubcoreMesh(
        core_axis_name="c", subcore_axis_name="s", num_cores=NC, num_subcores=NS
    )


def _worker_id():
    return lax.axis_index("s") * NC + lax.axis_index("c")


def _sc_degree(row3):
    return pl.kernel(
        _sc_degree_body,
        out_type=jax.ShapeDtypeStruct((NC, NP, HW), jnp.float32),
        mesh=_mesh(),
        scratch_types=[
            pltpu.VMEM((CH, K), jnp.int32),
            pltpu.VMEM((K, HW), jnp.float32),
            pltpu.VMEM_SHARED((NP, HW), jnp.float32),
        ],
    )(row3)


def _sc_degree_body(row_hbm, out_hbm, rowbuf, ones_b, degsh):
    cid = lax.axis_index("c")
    sid = lax.axis_index("s")
    wid = _worker_id()

    def _fill(val):
        def _f(i, _):
            ones_b[i, 0:HW] = jnp.full((HW,), val, jnp.float32)
            return 0
        lax.fori_loop(0, K, _f, 0)

    _fill(0.0)
    for t in range(RPS // K):
        pltpu.sync_copy(ones_b, degsh.at[pl.ds(sid * RPS + t * K, K)])
    _fill(1.0)
    plsc.subcore_barrier()

    pltpu.sync_copy(row_hbm.at[wid], rowbuf)

    def _chunk(c, _):
        pltpu.sync_copy(ones_b, degsh.at[rowbuf.at[c]], add=True)
        return 0

    lax.fori_loop(0, CH, _chunk, 0)
    plsc.subcore_barrier()
    pltpu.sync_copy(
        degsh.at[pl.ds(sid * RPS, RPS)], out_hbm.at[cid, pl.ds(sid * RPS, RPS)]
    )


KP = 64
CHP = EWP // KP
SR = 4
NSEC = CHP // SR


def _sc_propagate(y, row3p, col3p):
    return pl.kernel(
        _sc_propagate_body,
        out_type=jax.ShapeDtypeStruct((NC, NP, D), jnp.float32),
        mesh=_mesh(),
        scratch_types=[
            pltpu.VMEM((SR, KP), jnp.int32),
            pltpu.VMEM((SR, KP), jnp.int32),
            pltpu.VMEM((SR, KP, D), jnp.float32),
            pltpu.VMEM_SHARED((NP, D), jnp.float32),
        ],
    )(y, row3p, col3p)


def _sc_propagate_body(y_hbm, row_hbm, col_hbm, out_hbm,
                       rowsec, colsec, ybufs, acc):
    cid = lax.axis_index("c")
    sid = lax.axis_index("s")
    wid = _worker_id()

    def _zfill(i, _):
        for j in range(D // 16):
            ybufs[0, i, pl.ds(16 * j, 16)] = jnp.zeros((16,), jnp.float32)
        return 0

    lax.fori_loop(0, KP, _zfill, 0)
    for t in range(RPS // KP):
        pltpu.sync_copy(ybufs.at[0], acc.at[pl.ds(sid * RPS + t * KP, KP)])
    plsc.subcore_barrier()

    def _section(s, _):
        pltpu.sync_copy(row_hbm.at[wid, pl.ds(s * SR, SR)], rowsec)
        pltpu.sync_copy(col_hbm.at[wid, pl.ds(s * SR, SR)], colsec)

        @plsc.parallel_loop(0, SR, step=1, unroll=SR)
        def _chunk(r):
            pltpu.sync_copy(y_hbm.at[rowsec.at[r]], ybufs.at[r])
            pltpu.sync_copy(ybufs.at[r], acc.at[colsec.at[r]], add=True)

        return 0

    lax.fori_loop(0, NSEC, _section, 0)

    plsc.subcore_barrier()
    pltpu.sync_copy(
        acc.at[pl.ds(sid * RPS, RPS)], out_hbm.at[cid, pl.ds(sid * RPS, RPS)]
    )


def _prep_body(hist_ref, x_ref, y_ref):
    deg = (jnp.sum(hist_ref[0], axis=-1) + jnp.sum(hist_ref[1], axis=-1)) * (
        1.0 / HW
    ) + 1.0
    dis = lax.rsqrt(deg)
    y_ref[...] = dis[:, None] * x_ref[...]


def _tc_prep(hist, x_pad, nb=1280):
    grid = NP // nb
    return pl.pallas_call(
        _prep_body,
        grid=(grid,),
        in_specs=[
            pl.BlockSpec((NC, nb, HW), lambda i: (0, i, 0)),
            pl.BlockSpec((nb, D), lambda i: (i, 0)),
        ],
        out_specs=pl.BlockSpec((nb, D), lambda i: (i, 0)),
        out_shape=jax.ShapeDtypeStruct((NP, D), jnp.float32),
    )(hist, x_pad)


def _gru_body(hist_ref, acc_ref, y_ref, x_ref, wih_ref, whh_ref,
              bih_ref, bhh_ref, out_ref):
    deg = (jnp.sum(hist_ref[0], axis=-1) + jnp.sum(hist_ref[1], axis=-1)) * (
        1.0 / HW
    ) + 1.0
    dis = lax.rsqrt(deg)
    h = dis[:, None] * (acc_ref[0] + acc_ref[1] + y_ref[...])
    x = x_ref[...]
    gi = jnp.dot(h, wih_ref[...], preferred_element_type=jnp.float32) + bih_ref[...]
    gh = jnp.dot(x, whh_ref[...], preferred_element_type=jnp.float32) + bhh_ref[...]
    r = jax.nn.sigmoid(gi[:, 0:D] + gh[:, 0:D])
    z = jax.nn.sigmoid(gi[:, D:2 * D] + gh[:, D:2 * D])
    n = jnp.tanh(gi[:, 2 * D:3 * D] + r * gh[:, 2 * D:3 * D])
    out_ref[...] = (1.0 - z) * n + z * x


def _tc_gru(hist, acc, y, x, wih_t, whh_t, b_ih, b_hh, nb=2000):
    grid = N // nb
    return pl.pallas_call(
        _gru_body,
        grid=(grid,),
        in_specs=[
            pl.BlockSpec((NC, nb, HW), lambda i: (0, i, 0)),
            pl.BlockSpec((NC, nb, D), lambda i: (0, i, 0)),
            pl.BlockSpec((nb, D), lambda i: (i, 0)),
            pl.BlockSpec((nb, D), lambda i: (i, 0)),
            pl.BlockSpec((D, 3 * D), lambda i: (0, 0)),
            pl.BlockSpec((D, 3 * D), lambda i: (0, 0)),
            pl.BlockSpec((1, 3 * D), lambda i: (0, 0)),
            pl.BlockSpec((1, 3 * D), lambda i: (0, 0)),
        ],
        out_specs=pl.BlockSpec((nb, D), lambda i: (i, 0)),
        out_shape=jax.ShapeDtypeStruct((N, D), jnp.float32),
    )(hist, acc, y, x, wih_t, whh_t, b_ih, b_hh)


def kernel(x, edge_index, W_ih, W_hh, b_ih, b_hh):
    npad = EWP - E // NW
    w = jnp.arange(NW, dtype=jnp.int32)[:, None]
    j = jnp.arange(npad, dtype=jnp.int32)[None, :]
    fill_r = N + (w * 97 + j * 13) % (NP - N)
    fill_c = N + (w * 53 + j * 29) % (NP - N)
    rows2 = jnp.concatenate([edge_index[0].reshape(NW, E // NW), fill_r], axis=1)
    cols2 = jnp.concatenate([edge_index[1].reshape(NW, E // NW), fill_c], axis=1)
    row3 = rows2.reshape(NW, CH, K)
    row3p = rows2.reshape(NW, CHP, KP)
    col3p = cols2.reshape(NW, CHP, KP)
    x_pad = jnp.concatenate([x, jnp.zeros((NP - N, D), x.dtype)], axis=0)

    hist = _sc_degree(row3)
    y = _tc_prep(hist, x_pad)
    acc = _sc_propagate(y, row3p, col3p)
    return _tc_gru(hist, acc, y, x, W_ih.T, W_hh.T,
                   b_ih.reshape(1, 3 * D), b_hh.reshape(1, 3 * D))


# speedup vs baseline: 1.1931x; 1.1931x over previous
import jax
import jax.numpy as jnp
from jax import lax
from jax.experimental import pallas as pl
from jax.experimental.pallas import tpu as pltpu
from jax.experimental.pallas import tpu_sc as plsc

N = 10000
E = 320000
D = 128

NC = 2
NS = 16
NW = NC * NS
NP = 10240
RPS = NP // NS
K = 128
EWP = NP
CH = EWP // K
SECR = 8
SECS = CH // SECR
HW = 16


def _mesh():
    return plsc.VectorSubcoreMesh(
        core_axis_name="c", subcore_axis_name="s", num_cores=NC, num_subcores=NS
    )


def _worker_id():
    return lax.axis_index("s") * NC + lax.axis_index("c")


def _sc_degree(row3):
    return pl.kernel(
        _sc_degree_body,
        out_type=jax.ShapeDtypeStruct((NC, NP, HW), jnp.float32),
        mesh=_mesh(),
        scratch_types=[
            pltpu.VMEM((CH, K), jnp.int32),
            pltpu.VMEM((K, HW), jnp.float32),
            pltpu.VMEM_SHARED((NP, HW), jnp.float32),
        ],
    )(row3)


def _sc_degree_body(row_hbm, out_hbm, rowbuf, ones_b, degsh):
    cid = lax.axis_index("c")
    sid = lax.axis_index("s")
    wid = _worker_id()

    def _fill(val):
        def _f(i, _):
            ones_b[i, 0:HW] = jnp.full((HW,), val, jnp.float32)
            return 0
        lax.fori_loop(0, K, _f, 0)

    _fill(0.0)
    for t in range(RPS // K):
        pltpu.sync_copy(ones_b, degsh.at[pl.ds(sid * RPS + t * K, K)])
    _fill(1.0)
    plsc.subcore_barrier()

    pltpu.sync_copy(row_hbm.at[wid], rowbuf)

    def _chunk(c, _):
        pltpu.sync_copy(ones_b, degsh.at[rowbuf.at[c]], add=True)
        return 0

    lax.fori_loop(0, CH, _chunk, 0)
    plsc.subcore_barrier()
    pltpu.sync_copy(
        degsh.at[pl.ds(sid * RPS, RPS)], out_hbm.at[cid, pl.ds(sid * RPS, RPS)]
    )


KP = 128
CHP = EWP // KP
SR = 2
NSEC = CHP // SR


def _sc_propagate(y, row3p, col3p):
    return pl.kernel(
        _sc_propagate_body,
        out_type=jax.ShapeDtypeStruct((NC, NP, D), jnp.float32),
        mesh=_mesh(),
        scratch_types=[
            pltpu.VMEM((SR, KP), jnp.int32),
            pltpu.VMEM((SR, KP), jnp.int32),
            pltpu.VMEM((SR, KP, D), jnp.float32),
            pltpu.VMEM_SHARED((NP, D), jnp.float32),
        ],
    )(y, row3p, col3p)


def _sc_propagate_body(y_hbm, row_hbm, col_hbm, out_hbm,
                       rowsec, colsec, ybufs, acc):
    cid = lax.axis_index("c")
    sid = lax.axis_index("s")
    wid = _worker_id()

    def _zfill(i, _):
        for j in range(D // 16):
            ybufs[0, i, pl.ds(16 * j, 16)] = jnp.zeros((16,), jnp.float32)
        return 0

    lax.fori_loop(0, KP, _zfill, 0)
    for t in range(RPS // KP):
        pltpu.sync_copy(ybufs.at[0], acc.at[pl.ds(sid * RPS + t * KP, KP)])
    plsc.subcore_barrier()

    def _section(s, _):
        pltpu.sync_copy(row_hbm.at[wid, pl.ds(s * SR, SR)], rowsec)
        pltpu.sync_copy(col_hbm.at[wid, pl.ds(s * SR, SR)], colsec)

        @plsc.parallel_loop(0, SR, step=1, unroll=SR)
        def _chunk(r):
            pltpu.sync_copy(y_hbm.at[rowsec.at[r]], ybufs.at[r])
            pltpu.sync_copy(ybufs.at[r], acc.at[colsec.at[r]], add=True)

        return 0

    lax.fori_loop(0, NSEC, _section, 0)

    plsc.subcore_barrier()
    pltpu.sync_copy(
        acc.at[pl.ds(sid * RPS, RPS)], out_hbm.at[cid, pl.ds(sid * RPS, RPS)]
    )


def _prep_body(hist_ref, x_ref, y_ref):
    deg = (jnp.sum(hist_ref[0], axis=-1) + jnp.sum(hist_ref[1], axis=-1)) * (
        1.0 / HW
    ) + 1.0
    dis = lax.rsqrt(deg)
    y_ref[...] = dis[:, None] * x_ref[...]


def _tc_prep(hist, x_pad, nb=1280):
    grid = NP // nb
    return pl.pallas_call(
        _prep_body,
        grid=(grid,),
        in_specs=[
            pl.BlockSpec((NC, nb, HW), lambda i: (0, i, 0)),
            pl.BlockSpec((nb, D), lambda i: (i, 0)),
        ],
        out_specs=pl.BlockSpec((nb, D), lambda i: (i, 0)),
        out_shape=jax.ShapeDtypeStruct((NP, D), jnp.float32),
    )(hist, x_pad)


def _gru_body(hist_ref, acc_ref, y_ref, x_ref, wih_ref, whh_ref,
              bih_ref, bhh_ref, out_ref):
    deg = (jnp.sum(hist_ref[0], axis=-1) + jnp.sum(hist_ref[1], axis=-1)) * (
        1.0 / HW
    ) + 1.0
    dis = lax.rsqrt(deg)
    h = dis[:, None] * (acc_ref[0] + acc_ref[1] + y_ref[...])
    x = x_ref[...]
    gi = jnp.dot(h, wih_ref[...], preferred_element_type=jnp.float32) + bih_ref[...]
    gh = jnp.dot(x, whh_ref[...], preferred_element_type=jnp.float32) + bhh_ref[...]
    r = jax.nn.sigmoid(gi[:, 0:D] + gh[:, 0:D])
    z = jax.nn.sigmoid(gi[:, D:2 * D] + gh[:, D:2 * D])
    n = jnp.tanh(gi[:, 2 * D:3 * D] + r * gh[:, 2 * D:3 * D])
    out_ref[...] = (1.0 - z) * n + z * x


def _tc_gru(hist, acc, y, x, wih_t, whh_t, b_ih, b_hh, nb=2000):
    grid = N // nb
    return pl.pallas_call(
        _gru_body,
        grid=(grid,),
        in_specs=[
            pl.BlockSpec((NC, nb, HW), lambda i: (0, i, 0)),
            pl.BlockSpec((NC, nb, D), lambda i: (0, i, 0)),
            pl.BlockSpec((nb, D), lambda i: (i, 0)),
            pl.BlockSpec((nb, D), lambda i: (i, 0)),
            pl.BlockSpec((D, 3 * D), lambda i: (0, 0)),
            pl.BlockSpec((D, 3 * D), lambda i: (0, 0)),
            pl.BlockSpec((1, 3 * D), lambda i: (0, 0)),
            pl.BlockSpec((1, 3 * D), lambda i: (0, 0)),
        ],
        out_specs=pl.BlockSpec((nb, D), lambda i: (i, 0)),
        out_shape=jax.ShapeDtypeStruct((N, D), jnp.float32),
    )(hist, acc, y, x, wih_t, whh_t, b_ih, b_hh)


def kernel(x, edge_index, W_ih, W_hh, b_ih, b_hh):
    npad = EWP - E // NW
    w = jnp.arange(NW, dtype=jnp.int32)[:, None]
    j = jnp.arange(npad, dtype=jnp.int32)[None, :]
    fill_r = N + (w * 97 + j * 13) % (NP - N)
    fill_c = N + (w * 53 + j * 29) % (NP - N)
    rows2 = jnp.concatenate([edge_index[0].reshape(NW, E // NW), fill_r], axis=1)
    cols2 = jnp.concatenate([edge_index[1].reshape(NW, E // NW), fill_c], axis=1)
    row3 = rows2.reshape(NW, CH, K)
    row3p = rows2.reshape(NW, CHP, KP)
    col3p = cols2.reshape(NW, CHP, KP)
    x_pad = jnp.concatenate([x, jnp.zeros((NP - N, D), x.dtype)], axis=0)

    hist = _sc_degree(row3)
    y = _tc_prep(hist, x_pad)
    acc = _sc_propagate(y, row3p, col3p)
    return _tc_gru(hist, acc, y, x, W_ih.T, W_hh.T,
                   b_ih.reshape(1, 3 * D), b_hh.reshape(1, 3 * D))


# speedup vs baseline: 1.3638x; 1.1431x over previous
import jax
import jax.numpy as jnp
from jax import lax
from jax.experimental import pallas as pl
from jax.experimental.pallas import tpu as pltpu
from jax.experimental.pallas import tpu_sc as plsc

N = 10000
E = 320000
D = 128

NC = 2
NS = 16
NW = NC * NS
NP = 10240
RPS = NP // NS
K = 128
EWP = NP
CH = EWP // K
SECR = 8
SECS = CH // SECR
HW = 16


def _mesh():
    return plsc.VectorSubcoreMesh(
        core_axis_name="c", subcore_axis_name="s", num_cores=NC, num_subcores=NS
    )


def _worker_id():
    return lax.axis_index("s") * NC + lax.axis_index("c")


def _sc_degree(row3):
    return pl.kernel(
        _sc_degree_body,
        out_type=jax.ShapeDtypeStruct((NC, NP, HW), jnp.float32),
        mesh=_mesh(),
        scratch_types=[
            pltpu.VMEM((CH, K), jnp.int32),
            pltpu.VMEM((K, HW), jnp.float32),
            pltpu.VMEM_SHARED((NP, HW), jnp.float32),
        ],
    )(row3)


def _sc_degree_body(row_hbm, out_hbm, rowbuf, ones_b, degsh):
    cid = lax.axis_index("c")
    sid = lax.axis_index("s")
    wid = _worker_id()

    def _fill(val):
        def _f(i, _):
            ones_b[i, 0:HW] = jnp.full((HW,), val, jnp.float32)
            return 0
        lax.fori_loop(0, K, _f, 0)

    _fill(0.0)
    for t in range(RPS // K):
        pltpu.sync_copy(ones_b, degsh.at[pl.ds(sid * RPS + t * K, K)])
    _fill(1.0)
    plsc.subcore_barrier()

    pltpu.sync_copy(row_hbm.at[wid], rowbuf)

    def _chunk(c, _):
        pltpu.sync_copy(ones_b, degsh.at[rowbuf.at[c]], add=True)
        return 0

    lax.fori_loop(0, CH, _chunk, 0)
    plsc.subcore_barrier()
    pltpu.sync_copy(
        degsh.at[pl.ds(sid * RPS, RPS)], out_hbm.at[cid, pl.ds(sid * RPS, RPS)]
    )


def _sc_propagate(y, row3, col3):
    return pl.kernel(
        _sc_propagate_body,
        out_type=jax.ShapeDtypeStruct((NC, NP, D), jnp.float32),
        mesh=_mesh(),
        scratch_types=[
            pltpu.VMEM((CH, K), jnp.int32),
            pltpu.VMEM((CH, K), jnp.int32),
            pltpu.VMEM((K, D), jnp.float32),
            pltpu.VMEM_SHARED((NP, D), jnp.float32),
        ],
    )(y, row3, col3)


def _sc_propagate_body(y_hbm, row_hbm, col_hbm, out_hbm,
                       rowbuf, colbuf, ybuf, acc):
    cid = lax.axis_index("c")
    sid = lax.axis_index("s")
    wid = _worker_id()

    def _zfill(i, _):
        for j in range(D // 16):
            ybuf[i, pl.ds(16 * j, 16)] = jnp.zeros((16,), jnp.float32)
        return 0

    lax.fori_loop(0, K, _zfill, 0)
    for t in range(RPS // K):
        pltpu.sync_copy(ybuf, acc.at[pl.ds(sid * RPS + t * K, K)])
    plsc.subcore_barrier()

    pltpu.sync_copy(row_hbm.at[wid], rowbuf)
    pltpu.sync_copy(col_hbm.at[wid], colbuf)

    def _chunk(c, _):
        pltpu.sync_copy(y_hbm.at[rowbuf.at[c]], ybuf)
        pltpu.sync_copy(ybuf, acc.at[colbuf.at[c]], add=True)
        return 0

    lax.fori_loop(0, CH, _chunk, 0)

    plsc.subcore_barrier()
    pltpu.sync_copy(
        acc.at[pl.ds(sid * RPS, RPS)], out_hbm.at[cid, pl.ds(sid * RPS, RPS)]
    )


def _prep_body(hist_ref, x_ref, y_ref):
    deg = (jnp.sum(hist_ref[0], axis=-1) + jnp.sum(hist_ref[1], axis=-1)) * (
        1.0 / HW
    ) + 1.0
    dis = lax.rsqrt(deg)
    y_ref[...] = dis[:, None] * x_ref[...]


def _tc_prep(hist, x_pad, nb=1280):
    grid = NP // nb
    return pl.pallas_call(
        _prep_body,
        grid=(grid,),
        in_specs=[
            pl.BlockSpec((NC, nb, HW), lambda i: (0, i, 0)),
            pl.BlockSpec((nb, D), lambda i: (i, 0)),
        ],
        out_specs=pl.BlockSpec((nb, D), lambda i: (i, 0)),
        out_shape=jax.ShapeDtypeStruct((NP, D), jnp.float32),
    )(hist, x_pad)


def _gru_body(hist_ref, acc_ref, y_ref, x_ref, wih_ref, whh_ref,
              bih_ref, bhh_ref, out_ref):
    deg = (jnp.sum(hist_ref[0], axis=-1) + jnp.sum(hist_ref[1], axis=-1)) * (
        1.0 / HW
    ) + 1.0
    dis = lax.rsqrt(deg)
    h = dis[:, None] * (acc_ref[0] + acc_ref[1] + y_ref[...])
    x = x_ref[...]
    gi = jnp.dot(h, wih_ref[...], preferred_element_type=jnp.float32) + bih_ref[...]
    gh = jnp.dot(x, whh_ref[...], preferred_element_type=jnp.float32) + bhh_ref[...]
    r = jax.nn.sigmoid(gi[:, 0:D] + gh[:, 0:D])
    z = jax.nn.sigmoid(gi[:, D:2 * D] + gh[:, D:2 * D])
    n = jnp.tanh(gi[:, 2 * D:3 * D] + r * gh[:, 2 * D:3 * D])
    out_ref[...] = (1.0 - z) * n + z * x


def _tc_gru(hist, acc, y, x, wih_t, whh_t, b_ih, b_hh, nb=2000):
    grid = N // nb
    return pl.pallas_call(
        _gru_body,
        grid=(grid,),
        in_specs=[
            pl.BlockSpec((NC, nb, HW), lambda i: (0, i, 0)),
            pl.BlockSpec((NC, nb, D), lambda i: (0, i, 0)),
            pl.BlockSpec((nb, D), lambda i: (i, 0)),
            pl.BlockSpec((nb, D), lambda i: (i, 0)),
            pl.BlockSpec((D, 3 * D), lambda i: (0, 0)),
            pl.BlockSpec((D, 3 * D), lambda i: (0, 0)),
            pl.BlockSpec((1, 3 * D), lambda i: (0, 0)),
            pl.BlockSpec((1, 3 * D), lambda i: (0, 0)),
        ],
        out_specs=pl.BlockSpec((nb, D), lambda i: (i, 0)),
        out_shape=jax.ShapeDtypeStruct((N, D), jnp.float32),
    )(hist, acc, y, x, wih_t, whh_t, b_ih, b_hh)


def kernel(x, edge_index, W_ih, W_hh, b_ih, b_hh):
    npad = EWP - E // NW
    w = jnp.arange(NW, dtype=jnp.int32)[:, None]
    j = jnp.arange(npad, dtype=jnp.int32)[None, :]
    fill_r = N + (w * 97 + j * 13) % (NP - N)
    fill_c = N + (w * 53 + j * 29) % (NP - N)
    row3 = jnp.concatenate(
        [edge_index[0].reshape(NW, E // NW), fill_r], axis=1
    ).reshape(NW, CH, K)
    col3 = jnp.concatenate(
        [edge_index[1].reshape(NW, E // NW), fill_c], axis=1
    ).reshape(NW, CH, K)
    x_pad = jnp.concatenate([x, jnp.zeros((NP - N, D), x.dtype)], axis=0)

    hist = _sc_degree(row3)
    y = _tc_prep(hist, x_pad)
    acc = _sc_propagate(y, row3, col3)
    return _tc_gru(hist, acc, y, x, W_ih.T, W_hh.T,
                   b_ih.reshape(1, 3 * D), b_hh.reshape(1, 3 * D))
